# baseline (speedup 1.0000x reference)
import functools

import jax
import jax.numpy as jnp
import numpy as np
from jax.experimental import pallas as pl
from jax.experimental.pallas import tpu as pltpu

BN = 8
DIM = 192
HEADS = 8
NWIN = 7
TOPK = 4
HH = 56
WW = 56
HEAD_DIM = DIM // HEADS
PHD = 32
PDIM = HEADS * PHD
NREG = NWIN * NWIN
RH = HH // NWIN
SEG = RH * RH
HW = HH * WW
SCALE = DIM ** (-0.5)
NEG = -1e30
BF = jnp.bfloat16


def _qkv_kernel(x_ref, w_ref, b_ref, qkv_ref, qs_ref, ks_ref):
    w = w_ref[...]
    b = b_ref[...]
    for j in range(NWIN):
        x = x_ref[0, j]
        qkv = jnp.dot(w, x.astype(BF), preferred_element_type=jnp.float32) + b
        qkv_ref[0, j] = qkv.astype(BF)
        qs_ref[0, j] = jnp.sum(qkv[:PDIM], axis=1, keepdims=True)
        ks_ref[0, j] = jnp.sum(qkv[PDIM:2 * PDIM], axis=1, keepdims=True)


def _route_kernel(qs_ref, ks_ref, idx_ref):
    q_r = (qs_ref[0].reshape(NREG, PDIM) * (1.0 / SEG)).astype(BF)
    k_r = (ks_ref[0].reshape(NREG, PDIM) * (1.0 / SEG)).astype(BF)
    a = jax.lax.dot_general(q_r, k_r, (((1,), (1,)), ((), ())),
                            preferred_element_type=jnp.float32)
    cols = jax.lax.broadcasted_iota(jnp.int32, (NREG, NREG), 1)
    picks = []
    for _ in range(TOPK):
        m = jnp.max(a, axis=1, keepdims=True)
        cand = jnp.where(a == m, cols, NREG)
        sel = jnp.min(cand, axis=1, keepdims=True)
        picks.append(sel)
        a = jnp.where(cols == sel, NEG, a)
    idx_ref[0] = jnp.concatenate(picks, axis=1)


def _attn_kernel(idx_ref, q_ref, *refs):
    del idx_ref
    o_ref = refs[-1]
    k_refs = refs[:NWIN * TOPK]
    v_refs = refs[NWIN * TOPK:2 * NWIN * TOPK]
    for j in range(NWIN):
        q = q_ref[0, j].reshape(HEADS, PHD, SEG)
        kcat = jnp.concatenate(
            [k_refs[TOPK * j + t][0, 0].reshape(HEADS, PHD, SEG)
             for t in range(TOPK)], axis=2)
        vcat = jnp.concatenate(
            [v_refs[TOPK * j + t][0, 0].reshape(HEADS, PHD, SEG)
             for t in range(TOPK)], axis=2)
        a = jax.lax.dot_general(q, kcat, (((1,), (1,)), ((0,), (0,))),
                                preferred_element_type=jnp.float32) * SCALE
        m = jnp.max(a, axis=2, keepdims=True)
        e = jnp.exp(a - m)
        s = jnp.sum(e, axis=2, keepdims=True)
        prob = (e / s).astype(BF)
        o = jax.lax.dot_general(vcat, prob, (((2,), (2,)), ((0,), (0,))),
                                preferred_element_type=jnp.float32)
        o_ref[0, j] = o.astype(BF).reshape(PDIM, SEG)


def _lepe_out_kernel(vg_ref, att_ref, lw_ref, lb_ref, ow_ref, ob_ref, out_ref):
    v = vg_ref[0].astype(jnp.float32)
    zero = jnp.zeros((DIM, 64), jnp.float32)
    zp = jnp.concatenate([zero, v, zero], axis=1)
    col = jax.lax.rem(jax.lax.broadcasted_iota(jnp.int32, (DIM, HW), 1),
                      jnp.int32(WW))
    acc = jnp.zeros((DIM, HW), jnp.float32)
    for i in range(3):
        for j in range(3):
            off = 64 + (i - 1) * WW + (j - 1)
            tap = jax.lax.slice(zp, (0, off), (DIM, off + HW))
            if j == 0:
                tap = jnp.where(col == 0, 0.0, tap)
            elif j == 2:
                tap = jnp.where(col == WW - 1, 0.0, tap)
            wcol = jax.lax.slice(lw_ref[...], (0, 3 * i + j), (DIM, 3 * i + j + 1))
            acc = acc + tap * wcol
    y = att_ref[0].astype(jnp.float32) + acc + lb_ref[...]
    out = jnp.dot(ow_ref[...], y.astype(BF),
                  preferred_element_type=jnp.float32) + ob_ref[...]
    out_ref[0] = out


def kernel(x, qkv_w, qkv_b, lepe_w, lepe_b, out_w, out_b):
    n = x.shape[0]
    x4 = x.reshape(n, DIM, NWIN, RH, NWIN, RH).transpose(0, 2, 4, 1, 3, 5)
    x4 = x4.reshape(n, NREG, DIM, SEG)

    wp = jnp.pad(qkv_w.reshape(3, HEADS, HEAD_DIM, DIM),
                 ((0, 0), (0, 0), (0, PHD - HEAD_DIM), (0, 0)))
    wp = wp.reshape(3 * PDIM, DIM).astype(BF)
    bp = jnp.pad(qkv_b.reshape(3, HEADS, HEAD_DIM),
                 ((0, 0), (0, 0), (0, PHD - HEAD_DIM)))
    bp = bp.reshape(3 * PDIM, 1)

    qkv, qsum, ksum = pl.pallas_call(
        _qkv_kernel,
        grid=(n, NWIN),
        in_specs=[
            pl.BlockSpec((1, NWIN, DIM, SEG), lambda b, r: (b, r, 0, 0)),
            pl.BlockSpec((3 * PDIM, DIM), lambda b, r: (0, 0)),
            pl.BlockSpec((3 * PDIM, 1), lambda b, r: (0, 0)),
        ],
        out_specs=[
            pl.BlockSpec((1, NWIN, 3 * PDIM, SEG), lambda b, r: (b, r, 0, 0)),
            pl.BlockSpec((1, NWIN, PDIM, 1), lambda b, r: (b, r, 0, 0)),
            pl.BlockSpec((1, NWIN, PDIM, 1), lambda b, r: (b, r, 0, 0)),
        ],
        out_shape=[
            jax.ShapeDtypeStruct((n, NREG, 3 * PDIM, SEG), BF),
            jax.ShapeDtypeStruct((n, NREG, PDIM, 1), jnp.float32),
            jax.ShapeDtypeStruct((n, NREG, PDIM, 1), jnp.float32),
        ],
    )(x4, wp, bp)

    idx = pl.pallas_call(
        _route_kernel,
        grid=(n,),
        in_specs=[
            pl.BlockSpec((1, NREG, PDIM, 1), lambda b: (b, 0, 0, 0)),
            pl.BlockSpec((1, NREG, PDIM, 1), lambda b: (b, 0, 0, 0)),
        ],
        out_specs=pl.BlockSpec((1, NREG, TOPK), lambda b: (b, 0, 0)),
        out_shape=jax.ShapeDtypeStruct((n, NREG, TOPK), jnp.int32),
    )(qsum, ksum)

    def _kmap(j, t, sec):
        return lambda b, rg, idx: (b, idx[b, rg * NWIN + j, t], sec, 0)

    gather_specs = [pl.BlockSpec((1, 1, PDIM, SEG), _kmap(j, t, sec))
                    for sec in (1, 2)
                    for j in range(NWIN) for t in range(TOPK)]
    attn4 = pl.pallas_call(
        _attn_kernel,
        grid_spec=pltpu.PrefetchScalarGridSpec(
            num_scalar_prefetch=1,
            grid=(n, NWIN),
            in_specs=[
                pl.BlockSpec((1, NWIN, PDIM, SEG),
                             lambda b, rg, idx: (b, rg, 0, 0)),
            ] + gather_specs,
            out_specs=pl.BlockSpec((1, NWIN, PDIM, SEG),
                                   lambda b, rg, idx: (b, rg, 0, 0)),
        ),
        out_shape=jax.ShapeDtypeStruct((n, NREG, PDIM, SEG), BF),
    )(idx, *([qkv] * (1 + 2 * NWIN * TOPK)))

    def seq4_to_grid_flat(t):
        t = t.reshape(n, NREG, HEADS, PHD, SEG)[:, :, :, :HEAD_DIM, :]
        t = t.reshape(n, NWIN, NWIN, DIM, RH, RH).transpose(0, 3, 1, 4, 2, 5)
        return t.reshape(n, DIM, HW)

    v_grid = seq4_to_grid_flat(qkv[:, :, 2 * PDIM:, :])
    attn_grid = seq4_to_grid_flat(attn4)

    out = pl.pallas_call(
        _lepe_out_kernel,
        grid=(n,),
        in_specs=[
            pl.BlockSpec((1, DIM, HW), lambda b: (b, 0, 0)),
            pl.BlockSpec((1, DIM, HW), lambda b: (b, 0, 0)),
            pl.BlockSpec((DIM, 9), lambda b: (0, 0)),
            pl.BlockSpec((DIM, 1), lambda b: (0, 0)),
            pl.BlockSpec((DIM, DIM), lambda b: (0, 0)),
            pl.BlockSpec((DIM, 1), lambda b: (0, 0)),
        ],
        out_specs=pl.BlockSpec((1, DIM, HW), lambda b: (b, 0, 0)),
        out_shape=jax.ShapeDtypeStruct((n, DIM, HW), jnp.float32),
    )(v_grid, attn_grid, lepe_w.reshape(DIM, 9), lepe_b.reshape(DIM, 1),
      out_w.astype(BF), out_b.reshape(DIM, 1))

    return out.reshape(n, DIM, HH, WW)


# speedup vs baseline: 1.0409x; 1.0409x over previous
import functools

import jax
import jax.numpy as jnp
import numpy as np
from jax.experimental import pallas as pl
from jax.experimental.pallas import tpu as pltpu

BN = 8
DIM = 192
HEADS = 8
NWIN = 7
TOPK = 4
HH = 56
WW = 56
HEAD_DIM = DIM // HEADS
PHD = 32
PDIM = HEADS * PHD
NREG = NWIN * NWIN
RH = HH // NWIN
SEG = RH * RH
HW = HH * WW
SCALE = DIM ** (-0.5)
NEG = -1e30
BF = jnp.bfloat16


def _qkv_kernel(x_ref, w_ref, b_ref, qkv_ref, qs_ref, ks_ref):
    w = w_ref[...]
    b = b_ref[...]
    for j in range(NWIN):
        x = x_ref[0, j]
        qkv = jnp.dot(w, x.astype(BF), preferred_element_type=jnp.float32) + b
        qkv_ref[0, j] = qkv.astype(BF)
        ks_ref[0, j] = jnp.sum(qkv[:PDIM], axis=1, keepdims=True)
        qs_ref[0, j] = jnp.sum(qkv[2 * PDIM:], axis=1, keepdims=True)


def _route_kernel(qs_ref, ks_ref, idx_ref):
    q_r = (qs_ref[0].reshape(NREG, PDIM) * (1.0 / SEG)).astype(BF)
    k_r = (ks_ref[0].reshape(NREG, PDIM) * (1.0 / SEG)).astype(BF)
    a = jax.lax.dot_general(q_r, k_r, (((1,), (1,)), ((), ())),
                            preferred_element_type=jnp.float32)
    cols = jax.lax.broadcasted_iota(jnp.int32, (NREG, NREG), 1)
    picks = []
    for _ in range(TOPK):
        m = jnp.max(a, axis=1, keepdims=True)
        cand = jnp.where(a == m, cols, NREG)
        sel = jnp.min(cand, axis=1, keepdims=True)
        picks.append(sel)
        a = jnp.where(cols == sel, NEG, a)
    idx_ref[0] = jnp.concatenate(picks, axis=1)


def _attn_kernel(idx_ref, q_ref, *refs):
    del idx_ref
    o_ref = refs[-1]
    kv_refs = refs[:NWIN * TOPK]
    for j in range(NWIN):
        q = q_ref[0, j].reshape(HEADS, PHD, SEG)
        kvs = [kv_refs[TOPK * j + t][0, 0] for t in range(TOPK)]
        kcat = jnp.concatenate(
            [kv[:PDIM].reshape(HEADS, PHD, SEG) for kv in kvs], axis=2)
        vcat = jnp.concatenate(
            [kv[PDIM:].reshape(HEADS, PHD, SEG) for kv in kvs], axis=2)
        a = jax.lax.dot_general(q, kcat, (((1,), (1,)), ((0,), (0,))),
                                preferred_element_type=jnp.float32) * SCALE
        m = jnp.max(a, axis=2, keepdims=True)
        e = jnp.exp(a - m)
        s = jnp.sum(e, axis=2, keepdims=True)
        prob = (e * (1.0 / s)).astype(BF)
        o = jax.lax.dot_general(vcat, prob, (((2,), (2,)), ((0,), (0,))),
                                preferred_element_type=jnp.float32)
        o_ref[0, j] = o.astype(BF).reshape(PDIM, SEG)


def _lepe_out_kernel(vg_ref, att_ref, lw_ref, lb_ref, ow_ref, ob_ref, out_ref):
    v = vg_ref[0].astype(jnp.float32)
    zero = jnp.zeros((DIM, 64), jnp.float32)
    zp = jnp.concatenate([zero, v, zero], axis=1)
    col = jax.lax.rem(jax.lax.broadcasted_iota(jnp.int32, (DIM, HW), 1),
                      jnp.int32(WW))
    acc = jnp.zeros((DIM, HW), jnp.float32)
    for i in range(3):
        for j in range(3):
            off = 64 + (i - 1) * WW + (j - 1)
            tap = jax.lax.slice(zp, (0, off), (DIM, off + HW))
            if j == 0:
                tap = jnp.where(col == 0, 0.0, tap)
            elif j == 2:
                tap = jnp.where(col == WW - 1, 0.0, tap)
            wcol = jax.lax.slice(lw_ref[...], (0, 3 * i + j), (DIM, 3 * i + j + 1))
            acc = acc + tap * wcol
    y = att_ref[0].astype(jnp.float32) + acc + lb_ref[...]
    out = jnp.dot(ow_ref[...], y.astype(BF),
                  preferred_element_type=jnp.float32) + ob_ref[...]
    out_ref[0] = out


def kernel(x, qkv_w, qkv_b, lepe_w, lepe_b, out_w, out_b):
    n = x.shape[0]
    x4 = x.reshape(n, DIM, NWIN, RH, NWIN, RH).transpose(0, 2, 4, 1, 3, 5)
    x4 = x4.reshape(n, NREG, DIM, SEG)

    wp = jnp.pad(qkv_w.reshape(3, HEADS, HEAD_DIM, DIM),
                 ((0, 0), (0, 0), (0, PHD - HEAD_DIM), (0, 0)))[jnp.array([1, 2, 0])]
    wp = wp.reshape(3 * PDIM, DIM).astype(BF)
    bp = jnp.pad(qkv_b.reshape(3, HEADS, HEAD_DIM),
                 ((0, 0), (0, 0), (0, PHD - HEAD_DIM)))[jnp.array([1, 2, 0])]
    bp = bp.reshape(3 * PDIM, 1)

    qkv, qsum, ksum = pl.pallas_call(
        _qkv_kernel,
        grid=(n, NWIN),
        in_specs=[
            pl.BlockSpec((1, NWIN, DIM, SEG), lambda b, r: (b, r, 0, 0)),
            pl.BlockSpec((3 * PDIM, DIM), lambda b, r: (0, 0)),
            pl.BlockSpec((3 * PDIM, 1), lambda b, r: (0, 0)),
        ],
        out_specs=[
            pl.BlockSpec((1, NWIN, 3 * PDIM, SEG), lambda b, r: (b, r, 0, 0)),
            pl.BlockSpec((1, NWIN, PDIM, 1), lambda b, r: (b, r, 0, 0)),
            pl.BlockSpec((1, NWIN, PDIM, 1), lambda b, r: (b, r, 0, 0)),
        ],
        out_shape=[
            jax.ShapeDtypeStruct((n, NREG, 3 * PDIM, SEG), BF),
            jax.ShapeDtypeStruct((n, NREG, PDIM, 1), jnp.float32),
            jax.ShapeDtypeStruct((n, NREG, PDIM, 1), jnp.float32),
        ],
    )(x4, wp, bp)

    idx = pl.pallas_call(
        _route_kernel,
        grid=(n,),
        in_specs=[
            pl.BlockSpec((1, NREG, PDIM, 1), lambda b: (b, 0, 0, 0)),
            pl.BlockSpec((1, NREG, PDIM, 1), lambda b: (b, 0, 0, 0)),
        ],
        out_specs=pl.BlockSpec((1, NREG, TOPK), lambda b: (b, 0, 0)),
        out_shape=jax.ShapeDtypeStruct((n, NREG, TOPK), jnp.int32),
    )(qsum, ksum)

    def _kmap(j, t):
        return lambda b, rg, idx: (b, idx[b, rg * NWIN + j, t], 0, 0)

    gather_specs = [pl.BlockSpec((1, 1, 2 * PDIM, SEG), _kmap(j, t))
                    for j in range(NWIN) for t in range(TOPK)]
    attn4 = pl.pallas_call(
        _attn_kernel,
        grid_spec=pltpu.PrefetchScalarGridSpec(
            num_scalar_prefetch=1,
            grid=(n, NWIN),
            in_specs=[
                pl.BlockSpec((1, NWIN, PDIM, SEG),
                             lambda b, rg, idx: (b, rg, 2, 0)),
            ] + gather_specs,
            out_specs=pl.BlockSpec((1, NWIN, PDIM, SEG),
                                   lambda b, rg, idx: (b, rg, 0, 0)),
        ),
        out_shape=jax.ShapeDtypeStruct((n, NREG, PDIM, SEG), BF),
    )(idx, *([qkv] * (1 + NWIN * TOPK)))

    def seq4_to_grid_flat(t):
        t = t.reshape(n, NREG, HEADS, PHD, SEG)[:, :, :, :HEAD_DIM, :]
        t = t.reshape(n, NWIN, NWIN, DIM, RH, RH).transpose(0, 3, 1, 4, 2, 5)
        return t.reshape(n, DIM, HW)

    v_grid = seq4_to_grid_flat(qkv[:, :, PDIM:2 * PDIM, :])
    attn_grid = seq4_to_grid_flat(attn4)

    out = pl.pallas_call(
        _lepe_out_kernel,
        grid=(n,),
        in_specs=[
            pl.BlockSpec((1, DIM, HW), lambda b: (b, 0, 0)),
            pl.BlockSpec((1, DIM, HW), lambda b: (b, 0, 0)),
            pl.BlockSpec((DIM, 9), lambda b: (0, 0)),
            pl.BlockSpec((DIM, 1), lambda b: (0, 0)),
            pl.BlockSpec((DIM, DIM), lambda b: (0, 0)),
            pl.BlockSpec((DIM, 1), lambda b: (0, 0)),
        ],
        out_specs=pl.BlockSpec((1, DIM, HW), lambda b: (b, 0, 0)),
        out_shape=jax.ShapeDtypeStruct((n, DIM, HW), jnp.float32),
    )(v_grid, attn_grid, lepe_w.reshape(DIM, 9), lepe_b.reshape(DIM, 1),
      out_w.astype(BF), out_b.reshape(DIM, 1))

    return out.reshape(n, DIM, HH, WW)


# speedup vs baseline: 1.2411x; 1.1923x over previous
import functools

import jax
import jax.numpy as jnp
import numpy as np
from jax.experimental import pallas as pl
from jax.experimental.pallas import tpu as pltpu

BN = 8
DIM = 192
HEADS = 8
NWIN = 7
TOPK = 4
HH = 56
WW = 56
HEAD_DIM = DIM // HEADS
PHD = 32
PDIM = HEADS * PHD
NREG = NWIN * NWIN
RH = HH // NWIN
SEG = RH * RH
HW = HH * WW
SCALE = DIM ** (-0.5)
NEG = -1e30
BF = jnp.bfloat16


def _qkv_kernel(x_ref, w_ref, b_ref, qkv_ref, qs_ref, ks_ref):
    w = w_ref[...]
    b = b_ref[...]
    for j in range(NWIN):
        x = x_ref[0, j]
        qkv = jnp.dot(w, x.astype(BF), preferred_element_type=jnp.float32) + b
        qkv_ref[0, j] = qkv.astype(BF)
        ks_ref[0, j] = jnp.sum(qkv[:PDIM], axis=1, keepdims=True)
        qs_ref[0, j] = jnp.sum(qkv[2 * PDIM:], axis=1, keepdims=True)


def _route_kernel(qs_ref, ks_ref, idx_ref):
    q_r = (qs_ref[0].reshape(NREG, PDIM) * (1.0 / SEG)).astype(BF)
    k_r = (ks_ref[0].reshape(NREG, PDIM) * (1.0 / SEG)).astype(BF)
    a = jax.lax.dot_general(q_r, k_r, (((1,), (1,)), ((), ())),
                            preferred_element_type=jnp.float32)
    cols = jax.lax.broadcasted_iota(jnp.int32, (NREG, NREG), 1)
    picks = []
    for _ in range(TOPK):
        m = jnp.max(a, axis=1, keepdims=True)
        cand = jnp.where(a == m, cols, NREG)
        sel = jnp.min(cand, axis=1, keepdims=True)
        picks.append(sel)
        a = jnp.where(cols == sel, NEG, a)
    idx_ref[0] = jnp.concatenate(picks, axis=1)


def _attn_kernel(idx_ref, q_ref, *refs):
    del idx_ref
    o_ref = refs[-1]
    kv_refs = refs[:NWIN * TOPK]
    for j in range(NWIN):
        q = q_ref[0, j].reshape(HEADS, PHD, SEG)
        kvs = [kv_refs[TOPK * j + t][0, 0] for t in range(TOPK)]
        kcat = jnp.concatenate(
            [kv[:PDIM].reshape(HEADS, PHD, SEG) for kv in kvs], axis=2)
        vcat = jnp.concatenate(
            [kv[PDIM:].reshape(HEADS, PHD, SEG) for kv in kvs], axis=2)
        a = jax.lax.dot_general(q, kcat, (((1,), (1,)), ((0,), (0,))),
                                preferred_element_type=jnp.float32) * SCALE
        e = jnp.exp(a).astype(BF)
        o_raw = jax.lax.dot_general(vcat, e, (((2,), (2,)), ((0,), (0,))),
                                    preferred_element_type=jnp.float32)
        rs = 1.0 / o_raw[:, HEAD_DIM:HEAD_DIM + 1, :]
        o_ref[0, j] = (o_raw * rs).astype(BF).reshape(PDIM, SEG)


def _lepe_out_kernel(vg_ref, att_ref, lw_ref, lb_ref, ow_ref, ob_ref, out_ref):
    v = vg_ref[0].astype(jnp.float32)
    zero = jnp.zeros((DIM, 64), jnp.float32)
    zp = jnp.concatenate([zero, v, zero], axis=1)
    col = jax.lax.rem(jax.lax.broadcasted_iota(jnp.int32, (DIM, HW), 1),
                      jnp.int32(WW))
    acc = jnp.zeros((DIM, HW), jnp.float32)
    for i in range(3):
        for j in range(3):
            off = 64 + (i - 1) * WW + (j - 1)
            tap = jax.lax.slice(zp, (0, off), (DIM, off + HW))
            if j == 0:
                tap = jnp.where(col == 0, 0.0, tap)
            elif j == 2:
                tap = jnp.where(col == WW - 1, 0.0, tap)
            wcol = jax.lax.slice(lw_ref[...], (0, 3 * i + j), (DIM, 3 * i + j + 1))
            acc = acc + tap * wcol
    y = att_ref[0].astype(jnp.float32) + acc + lb_ref[...]
    out = jnp.dot(ow_ref[...], y.astype(BF),
                  preferred_element_type=jnp.float32) + ob_ref[...]
    out_ref[0] = out


def kernel(x, qkv_w, qkv_b, lepe_w, lepe_b, out_w, out_b):
    n = x.shape[0]
    x4 = x.reshape(n, DIM, NWIN, RH, NWIN, RH).transpose(0, 2, 4, 1, 3, 5)
    x4 = x4.reshape(n, NREG, DIM, SEG)

    wp = jnp.pad(qkv_w.reshape(3, HEADS, HEAD_DIM, DIM),
                 ((0, 0), (0, 0), (0, PHD - HEAD_DIM), (0, 0)))[jnp.array([1, 2, 0])]
    wp = wp.reshape(3 * PDIM, DIM).astype(BF)
    bp = jnp.pad(qkv_b.reshape(3, HEADS, HEAD_DIM),
                 ((0, 0), (0, 0), (0, PHD - HEAD_DIM)))[jnp.array([1, 2, 0])]
    bp = bp.at[1, :, HEAD_DIM].set(1.0)
    bp = bp.reshape(3 * PDIM, 1)

    qkv, qsum, ksum = pl.pallas_call(
        _qkv_kernel,
        grid=(n, NWIN),
        in_specs=[
            pl.BlockSpec((1, NWIN, DIM, SEG), lambda b, r: (b, r, 0, 0)),
            pl.BlockSpec((3 * PDIM, DIM), lambda b, r: (0, 0)),
            pl.BlockSpec((3 * PDIM, 1), lambda b, r: (0, 0)),
        ],
        out_specs=[
            pl.BlockSpec((1, NWIN, 3 * PDIM, SEG), lambda b, r: (b, r, 0, 0)),
            pl.BlockSpec((1, NWIN, PDIM, 1), lambda b, r: (b, r, 0, 0)),
            pl.BlockSpec((1, NWIN, PDIM, 1), lambda b, r: (b, r, 0, 0)),
        ],
        out_shape=[
            jax.ShapeDtypeStruct((n, NREG, 3 * PDIM, SEG), BF),
            jax.ShapeDtypeStruct((n, NREG, PDIM, 1), jnp.float32),
            jax.ShapeDtypeStruct((n, NREG, PDIM, 1), jnp.float32),
        ],
    )(x4, wp, bp)

    idx = pl.pallas_call(
        _route_kernel,
        grid=(n,),
        in_specs=[
            pl.BlockSpec((1, NREG, PDIM, 1), lambda b: (b, 0, 0, 0)),
            pl.BlockSpec((1, NREG, PDIM, 1), lambda b: (b, 0, 0, 0)),
        ],
        out_specs=pl.BlockSpec((1, NREG, TOPK), lambda b: (b, 0, 0)),
        out_shape=jax.ShapeDtypeStruct((n, NREG, TOPK), jnp.int32),
    )(qsum, ksum)

    def _kmap(j, t):
        return lambda b, rg, idx: (b, idx[b, rg * NWIN + j, t], 0, 0)

    gather_specs = [pl.BlockSpec((1, 1, 2 * PDIM, SEG), _kmap(j, t))
                    for j in range(NWIN) for t in range(TOPK)]
    attn4 = pl.pallas_call(
        _attn_kernel,
        grid_spec=pltpu.PrefetchScalarGridSpec(
            num_scalar_prefetch=1,
            grid=(n, NWIN),
            in_specs=[
                pl.BlockSpec((1, NWIN, PDIM, SEG),
                             lambda b, rg, idx: (b, rg, 2, 0)),
            ] + gather_specs,
            out_specs=pl.BlockSpec((1, NWIN, PDIM, SEG),
                                   lambda b, rg, idx: (b, rg, 0, 0)),
        ),
        out_shape=jax.ShapeDtypeStruct((n, NREG, PDIM, SEG), BF),
    )(idx, *([qkv] * (1 + NWIN * TOPK)))

    def seq4_to_grid_flat(t):
        t = t.reshape(n, NREG, HEADS, PHD, SEG)[:, :, :, :HEAD_DIM, :]
        t = t.reshape(n, NWIN, NWIN, DIM, RH, RH).transpose(0, 3, 1, 4, 2, 5)
        return t.reshape(n, DIM, HW)

    v_grid = seq4_to_grid_flat(qkv[:, :, PDIM:2 * PDIM, :])
    attn_grid = seq4_to_grid_flat(attn4)

    out = pl.pallas_call(
        _lepe_out_kernel,
        grid=(n,),
        in_specs=[
            pl.BlockSpec((1, DIM, HW), lambda b: (b, 0, 0)),
            pl.BlockSpec((1, DIM, HW), lambda b: (b, 0, 0)),
            pl.BlockSpec((DIM, 9), lambda b: (0, 0)),
            pl.BlockSpec((DIM, 1), lambda b: (0, 0)),
            pl.BlockSpec((DIM, DIM), lambda b: (0, 0)),
            pl.BlockSpec((DIM, 1), lambda b: (0, 0)),
        ],
        out_specs=pl.BlockSpec((1, DIM, HW), lambda b: (b, 0, 0)),
        out_shape=jax.ShapeDtypeStruct((n, DIM, HW), jnp.float32),
    )(v_grid, attn_grid, lepe_w.reshape(DIM, 9), lepe_b.reshape(DIM, 1),
      out_w.astype(BF), out_b.reshape(DIM, 1))

    return out.reshape(n, DIM, HH, WW)
